# baseline (speedup 1.0000x reference)
import functools

import jax
import jax.numpy as jnp
from jax import lax
from jax.experimental import pallas as pl
from jax.experimental.pallas import tpu as pltpu
from jax.experimental.pallas import tpu_sc as plsc

T, D, E, H = 8192, 768, 16, 128
TB = 512
B = 256
NB = T // B
G = NB + E
NC, NS = 2, 16
NW = NC * NS
PW = T // NW
CH = 128


def _gate_body(x_ref, wg_ref, bg_ref, eidx_ref, rank_ref, cnt_ref, off_ref,
               acc_ref):
    i = pl.program_id(0)

    @pl.when(i == 0)
    def _():
        acc_ref[...] = jnp.zeros_like(acc_ref)

    logits = lax.dot_general(
        x_ref[...], wg_ref[...], (((1,), (1,)), ((), ())),
        preferred_element_type=jnp.float32,
    ) + bg_ref[...]
    m = jnp.max(logits, axis=1, keepdims=True)
    iota_e = lax.broadcasted_iota(jnp.int32, (TB, E), 1)
    eidx = jnp.min(jnp.where(logits == m, iota_e, E), axis=1, keepdims=True)
    onehot = (iota_e == eidx).astype(jnp.float32)

    prev = acc_ref[...]
    rank_prev = jnp.sum(onehot * prev, axis=1, keepdims=True)
    r_iota = lax.broadcasted_iota(jnp.int32, (TB, TB), 0)
    c_iota = lax.broadcasted_iota(jnp.int32, (TB, TB), 1)
    tri = (c_iota < r_iota).astype(jnp.float32)
    cnt_in = lax.dot_general(
        tri, onehot, (((1,), (0,)), ((), ())),
        preferred_element_type=jnp.float32,
    )
    rank_in = jnp.sum(cnt_in * onehot, axis=1, keepdims=True)

    eidx_ref[...] = eidx
    rank_ref[...] = (rank_prev + rank_in).astype(jnp.int32)
    new = prev + jnp.sum(onehot, axis=0, keepdims=True)
    acc_ref[...] = new
    cnt_ref[...] = new.astype(jnp.int32)
    r16 = lax.broadcasted_iota(jnp.int32, (E, E), 0)
    c16 = lax.broadcasted_iota(jnp.int32, (E, E), 1)
    upper = (r16 < c16).astype(jnp.float32)
    off_ref[...] = lax.dot_general(
        new, upper, (((1,), (0,)), ((), ())),
        preferred_element_type=jnp.float32,
        precision=lax.Precision.HIGHEST,
    ).astype(jnp.int32)


_gate = pl.pallas_call(
    _gate_body,
    grid=(T // TB,),
    in_specs=[
        pl.BlockSpec((TB, D), lambda i: (i, 0)),
        pl.BlockSpec((E, D), lambda i: (0, 0)),
        pl.BlockSpec((1, E), lambda i: (0, 0)),
    ],
    out_specs=[
        pl.BlockSpec((TB, 1), lambda i: (i, 0)),
        pl.BlockSpec((TB, 1), lambda i: (i, 0)),
        pl.BlockSpec((1, E), lambda i: (0, 0)),
        pl.BlockSpec((1, E), lambda i: (0, 0)),
    ],
    out_shape=[
        jax.ShapeDtypeStruct((T, 1), jnp.int32),
        jax.ShapeDtypeStruct((T, 1), jnp.int32),
        jax.ShapeDtypeStruct((1, E), jnp.int32),
        jax.ShapeDtypeStruct((1, E), jnp.int32),
    ],
    scratch_shapes=[pltpu.VMEM((1, E), jnp.float32)],
)


@functools.lru_cache(maxsize=None)
def _make_sc_dispatch():
    mesh = plsc.VectorSubcoreMesh(core_axis_name="c", subcore_axis_name="s",
                                  num_cores=NC, num_subcores=NS)

    @functools.partial(
        pl.kernel,
        out_type=[
            jax.ShapeDtypeStruct((T, D), jnp.float32),
            jax.ShapeDtypeStruct((T,), jnp.int32),
        ],
        mesh=mesh,
        scratch_types=[
            pltpu.VMEM((E,), jnp.int32),
            pltpu.VMEM((CH,), jnp.int32),
            pltpu.VMEM((CH,), jnp.int32),
            pltpu.VMEM((CH, D), jnp.float32),
            pltpu.SemaphoreType.DMA,
        ],
        compiler_params=pltpu.CompilerParams(needs_layout_passes=False),
    )
    def _sc_dispatch(x_hbm, e_hbm, r_hbm, off_hbm, xs_hbm, p_hbm,
                     off_v, e_v, p_v, rows_v, sem):
        wid = lax.axis_index("s") * NC + lax.axis_index("c")
        base = wid * PW
        pltpu.sync_copy(off_hbm, off_v)
        for c in range(PW // CH):
            cbase = base + c * CH
            pltpu.sync_copy(e_hbm.at[pl.ds(cbase, CH)], e_v)
            pltpu.sync_copy(r_hbm.at[pl.ds(cbase, CH)], p_v)
            for j in range(CH // 16):
                sl = pl.ds(j * 16, 16)
                p_v[sl] = p_v[sl] + plsc.load_gather(off_v, [e_v[sl]])
            pltpu.sync_copy(x_hbm.at[pl.ds(cbase, CH)], rows_v)
            pltpu.async_copy(rows_v, xs_hbm.at[p_v], sem).wait()
            pltpu.sync_copy(p_v, p_hbm.at[pl.ds(cbase, CH)])

    return _sc_dispatch


def _mlp_body(tm_ref, te_ref, ts_ref, tn_ref, first_ref,
              xs_ref, w1_ref, b1_ref, w2_ref, b2_ref, w3_ref, out_ref):
    i = pl.program_id(0)
    h1 = jnp.maximum(
        lax.dot_general(
            xs_ref[...], w1_ref[0], (((1,), (1,)), ((), ())),
            preferred_element_type=jnp.float32,
        ) + b1_ref[0], 0.0)
    h2 = jnp.maximum(
        lax.dot_general(
            h1, w2_ref[0], (((1,), (1,)), ((), ())),
            preferred_element_type=jnp.float32,
        ) + b2_ref[0], 0.0)
    o = jnp.sum(h2 * w3_ref[0], axis=1, keepdims=True)
    gidx = tm_ref[i] * B + lax.broadcasted_iota(jnp.int32, (B, 1), 0)
    val = jnp.where((gidx >= ts_ref[i]) & (gidx < tn_ref[i]), o, 0.0)

    @pl.when(first_ref[i] != 0)
    def _():
        out_ref[...] = val

    @pl.when(first_ref[i] == 0)
    def _():
        out_ref[...] = out_ref[...] + val


_mlp = pl.pallas_call(
    _mlp_body,
    grid_spec=pltpu.PrefetchScalarGridSpec(
        num_scalar_prefetch=5,
        grid=(G,),
        in_specs=[
            pl.BlockSpec((B, D), lambda i, *r: (r[0][i], 0)),
            pl.BlockSpec((1, H, D), lambda i, *r: (r[1][i], 0, 0)),
            pl.BlockSpec((1, 1, H), lambda i, *r: (r[1][i], 0, 0)),
            pl.BlockSpec((1, H, H), lambda i, *r: (r[1][i], 0, 0)),
            pl.BlockSpec((1, 1, H), lambda i, *r: (r[1][i], 0, 0)),
            pl.BlockSpec((1, 1, H), lambda i, *r: (r[1][i], 0, 0)),
        ],
        out_specs=pl.BlockSpec((B, 1), lambda i, *r: (r[0][i], 0)),
    ),
    out_shape=jax.ShapeDtypeStruct((T, 1), jnp.float32),
)


@functools.lru_cache(maxsize=None)
def _make_sc_combine():
    mesh = plsc.VectorSubcoreMesh(core_axis_name="c", subcore_axis_name="s",
                                  num_cores=NC, num_subcores=NS)

    @functools.partial(
        pl.kernel,
        out_type=jax.ShapeDtypeStruct((T,), jnp.float32),
        mesh=mesh,
        scratch_types=[
            pltpu.VMEM((T,), jnp.float32),
            pltpu.VMEM((E,), jnp.float32),
            pltpu.VMEM((PW,), jnp.int32),
            pltpu.VMEM((PW,), jnp.int32),
            pltpu.VMEM((PW,), jnp.float32),
        ],
        compiler_params=pltpu.CompilerParams(needs_layout_passes=False),
    )
    def _sc_combine(o_hbm, p_hbm, e_hbm, b3_hbm, out_hbm,
                    o_v, b3_v, p_v, e_v, res_v):
        wid = lax.axis_index("s") * NC + lax.axis_index("c")
        base = wid * PW
        pltpu.sync_copy(o_hbm, o_v)
        pltpu.sync_copy(b3_hbm, b3_v)
        pltpu.sync_copy(p_hbm.at[pl.ds(base, PW)], p_v)
        pltpu.sync_copy(e_hbm.at[pl.ds(base, PW)], e_v)
        for j in range(PW // 16):
            sl = pl.ds(j * 16, 16)
            res_v[sl] = (plsc.load_gather(o_v, [p_v[sl]])
                         + plsc.load_gather(b3_v, [e_v[sl]]))
        pltpu.sync_copy(res_v, out_hbm.at[pl.ds(base, PW)])

    return _sc_combine


def _tile_metadata(counts):
    zero = jnp.zeros((1,), jnp.int32)
    offsets = jnp.concatenate([zero, jnp.cumsum(counts, dtype=jnp.int32)])
    m_first = offsets[:E] // B
    m_last = jnp.maximum(offsets[1:] - 1, 0) // B
    n_e = jnp.where(counts > 0, m_last - m_first + 1, 0).astype(jnp.int32)
    start = jnp.concatenate([zero, jnp.cumsum(n_e, dtype=jnp.int32)])
    total = start[E]
    ii = jnp.arange(G, dtype=jnp.int32)
    tile_e = jnp.sum((ii[:, None] >= start[None, 1:E + 1]).astype(jnp.int32),
                     axis=1)
    valid = ii < total
    tile_e = jnp.where(valid, jnp.minimum(tile_e, E - 1), E - 1)
    tile_m = jnp.where(valid, m_first[tile_e] + (ii - start[tile_e]), NB - 1)
    ts = jnp.where(valid, jnp.maximum(offsets[tile_e], tile_m * B), 0)
    tn = jnp.where(valid, jnp.minimum(offsets[tile_e + 1], (tile_m + 1) * B), 0)
    first = jnp.concatenate([jnp.ones((1,), jnp.int32),
                             (tile_m[1:] != tile_m[:-1]).astype(jnp.int32)])
    return tile_m, tile_e, ts, tn, first


def kernel(x, Wg, bg, W1, b1, W2, b2, W3, b3):
    eidx, rank, counts, offs = _gate(x, Wg, bg.reshape(1, E))
    e_flat = eidx.reshape(T)
    c_flat = counts.reshape(E)
    xs, p = _make_sc_dispatch()(x, e_flat, rank.reshape(T), offs.reshape(E))
    tile_m, tile_e, ts, tn, first = _tile_metadata(c_flat)
    o = _mlp(tile_m, tile_e, ts, tn, first,
             xs, W1, b1.reshape(E, 1, H), W2, b2.reshape(E, 1, H), W3)
    out = _make_sc_combine()(o.reshape(T), p, e_flat, b3.reshape(E))
    return out.reshape(T, 1)


# speedup vs baseline: 1.0783x; 1.0783x over previous
import functools

import jax
import jax.numpy as jnp
from jax import lax
from jax.experimental import pallas as pl
from jax.experimental.pallas import tpu as pltpu
from jax.experimental.pallas import tpu_sc as plsc

T, D, E, H = 8192, 768, 16, 128
TB = 512
B = 512
NB = T // B
G = NB + E
NC, NS = 2, 16
NW = NC * NS
PW = T // NW
CH = 128


def _gate_body(x_ref, wg_ref, bg_ref, eidx_ref, rank_ref, cnt_ref, off_ref,
               acc_ref):
    i = pl.program_id(0)

    @pl.when(i == 0)
    def _():
        acc_ref[...] = jnp.zeros_like(acc_ref)

    logits = lax.dot_general(
        x_ref[...], wg_ref[...], (((1,), (1,)), ((), ())),
        preferred_element_type=jnp.float32,
    ) + bg_ref[...]
    m = jnp.max(logits, axis=1, keepdims=True)
    iota_e = lax.broadcasted_iota(jnp.int32, (TB, E), 1)
    eidx = jnp.min(jnp.where(logits == m, iota_e, E), axis=1, keepdims=True)
    onehot = (iota_e == eidx).astype(jnp.float32)

    prev = acc_ref[...]
    rank_prev = jnp.sum(onehot * prev, axis=1, keepdims=True)
    r_iota = lax.broadcasted_iota(jnp.int32, (TB, TB), 0)
    c_iota = lax.broadcasted_iota(jnp.int32, (TB, TB), 1)
    tri = (c_iota < r_iota).astype(jnp.float32)
    cnt_in = lax.dot_general(
        tri, onehot, (((1,), (0,)), ((), ())),
        preferred_element_type=jnp.float32,
    )
    rank_in = jnp.sum(cnt_in * onehot, axis=1, keepdims=True)

    eidx_ref[...] = eidx
    rank_ref[...] = (rank_prev + rank_in).astype(jnp.int32)
    new = prev + jnp.sum(onehot, axis=0, keepdims=True)
    acc_ref[...] = new
    cnt_ref[...] = new.astype(jnp.int32)
    r16 = lax.broadcasted_iota(jnp.int32, (E, E), 0)
    c16 = lax.broadcasted_iota(jnp.int32, (E, E), 1)
    upper = (r16 < c16).astype(jnp.float32)
    off_ref[...] = lax.dot_general(
        new, upper, (((1,), (0,)), ((), ())),
        preferred_element_type=jnp.float32,
        precision=lax.Precision.HIGHEST,
    ).astype(jnp.int32)


_gate = pl.pallas_call(
    _gate_body,
    grid=(T // TB,),
    in_specs=[
        pl.BlockSpec((TB, D), lambda i: (i, 0)),
        pl.BlockSpec((E, D), lambda i: (0, 0)),
        pl.BlockSpec((1, E), lambda i: (0, 0)),
    ],
    out_specs=[
        pl.BlockSpec((TB, 1), lambda i: (i, 0)),
        pl.BlockSpec((TB, 1), lambda i: (i, 0)),
        pl.BlockSpec((1, E), lambda i: (0, 0)),
        pl.BlockSpec((1, E), lambda i: (0, 0)),
    ],
    out_shape=[
        jax.ShapeDtypeStruct((T, 1), jnp.int32),
        jax.ShapeDtypeStruct((T, 1), jnp.int32),
        jax.ShapeDtypeStruct((1, E), jnp.int32),
        jax.ShapeDtypeStruct((1, E), jnp.int32),
    ],
    scratch_shapes=[pltpu.VMEM((1, E), jnp.float32)],
)


@functools.lru_cache(maxsize=None)
def _make_sc_dispatch():
    mesh = plsc.VectorSubcoreMesh(core_axis_name="c", subcore_axis_name="s",
                                  num_cores=NC, num_subcores=NS)

    @functools.partial(
        pl.kernel,
        out_type=[
            jax.ShapeDtypeStruct((T, D), jnp.float32),
            jax.ShapeDtypeStruct((T,), jnp.int32),
        ],
        mesh=mesh,
        scratch_types=[
            pltpu.VMEM((E,), jnp.int32),
            pltpu.VMEM((CH,), jnp.int32),
            pltpu.VMEM((CH,), jnp.int32),
            pltpu.VMEM((CH, D), jnp.float32),
            pltpu.SemaphoreType.DMA,
        ],
        compiler_params=pltpu.CompilerParams(needs_layout_passes=False),
    )
    def _sc_dispatch(x_hbm, e_hbm, r_hbm, off_hbm, xs_hbm, p_hbm,
                     off_v, e_v, p_v, rows_v, sem):
        wid = lax.axis_index("s") * NC + lax.axis_index("c")
        base = wid * PW
        pltpu.sync_copy(off_hbm, off_v)
        for c in range(PW // CH):
            cbase = base + c * CH
            pltpu.sync_copy(e_hbm.at[pl.ds(cbase, CH)], e_v)
            pltpu.sync_copy(r_hbm.at[pl.ds(cbase, CH)], p_v)
            for j in range(CH // 16):
                sl = pl.ds(j * 16, 16)
                p_v[sl] = p_v[sl] + plsc.load_gather(off_v, [e_v[sl]])
            pltpu.sync_copy(x_hbm.at[pl.ds(cbase, CH)], rows_v)
            pltpu.async_copy(rows_v, xs_hbm.at[p_v], sem).wait()
            pltpu.sync_copy(p_v, p_hbm.at[pl.ds(cbase, CH)])

    return _sc_dispatch


def _mlp_body(tm_ref, te_ref, ts_ref, tn_ref, first_ref,
              xs_ref, w1_ref, b1_ref, w2_ref, b2_ref, w3_ref, out_ref):
    i = pl.program_id(0)
    h1 = jnp.maximum(
        lax.dot_general(
            xs_ref[...], w1_ref[0], (((1,), (1,)), ((), ())),
            preferred_element_type=jnp.float32,
        ) + b1_ref[0], 0.0)
    h2 = jnp.maximum(
        lax.dot_general(
            h1, w2_ref[0], (((1,), (1,)), ((), ())),
            preferred_element_type=jnp.float32,
        ) + b2_ref[0], 0.0)
    o = jnp.sum(h2 * w3_ref[0], axis=1, keepdims=True)
    gidx = tm_ref[i] * B + lax.broadcasted_iota(jnp.int32, (B, 1), 0)
    val = jnp.where((gidx >= ts_ref[i]) & (gidx < tn_ref[i]), o, 0.0)

    @pl.when(first_ref[i] != 0)
    def _():
        out_ref[...] = val

    @pl.when(first_ref[i] == 0)
    def _():
        out_ref[...] = out_ref[...] + val


_mlp = pl.pallas_call(
    _mlp_body,
    grid_spec=pltpu.PrefetchScalarGridSpec(
        num_scalar_prefetch=5,
        grid=(G,),
        in_specs=[
            pl.BlockSpec((B, D), lambda i, *r: (r[0][i], 0)),
            pl.BlockSpec((1, H, D), lambda i, *r: (r[1][i], 0, 0)),
            pl.BlockSpec((1, 1, H), lambda i, *r: (r[1][i], 0, 0)),
            pl.BlockSpec((1, H, H), lambda i, *r: (r[1][i], 0, 0)),
            pl.BlockSpec((1, 1, H), lambda i, *r: (r[1][i], 0, 0)),
            pl.BlockSpec((1, 1, H), lambda i, *r: (r[1][i], 0, 0)),
        ],
        out_specs=pl.BlockSpec((B, 1), lambda i, *r: (r[0][i], 0)),
    ),
    out_shape=jax.ShapeDtypeStruct((T, 1), jnp.float32),
)


@functools.lru_cache(maxsize=None)
def _make_sc_combine():
    mesh = plsc.VectorSubcoreMesh(core_axis_name="c", subcore_axis_name="s",
                                  num_cores=NC, num_subcores=NS)

    @functools.partial(
        pl.kernel,
        out_type=jax.ShapeDtypeStruct((T,), jnp.float32),
        mesh=mesh,
        scratch_types=[
            pltpu.VMEM((T,), jnp.float32),
            pltpu.VMEM((E,), jnp.float32),
            pltpu.VMEM((PW,), jnp.int32),
            pltpu.VMEM((PW,), jnp.int32),
            pltpu.VMEM((PW,), jnp.float32),
        ],
        compiler_params=pltpu.CompilerParams(needs_layout_passes=False),
    )
    def _sc_combine(o_hbm, p_hbm, e_hbm, b3_hbm, out_hbm,
                    o_v, b3_v, p_v, e_v, res_v):
        wid = lax.axis_index("s") * NC + lax.axis_index("c")
        base = wid * PW
        pltpu.sync_copy(o_hbm, o_v)
        pltpu.sync_copy(b3_hbm, b3_v)
        pltpu.sync_copy(p_hbm.at[pl.ds(base, PW)], p_v)
        pltpu.sync_copy(e_hbm.at[pl.ds(base, PW)], e_v)
        for j in range(PW // 16):
            sl = pl.ds(j * 16, 16)
            res_v[sl] = (plsc.load_gather(o_v, [p_v[sl]])
                         + plsc.load_gather(b3_v, [e_v[sl]]))
        pltpu.sync_copy(res_v, out_hbm.at[pl.ds(base, PW)])

    return _sc_combine


def _tile_metadata(counts):
    zero = jnp.zeros((1,), jnp.int32)
    offsets = jnp.concatenate([zero, jnp.cumsum(counts, dtype=jnp.int32)])
    m_first = offsets[:E] // B
    m_last = jnp.maximum(offsets[1:] - 1, 0) // B
    n_e = jnp.where(counts > 0, m_last - m_first + 1, 0).astype(jnp.int32)
    start = jnp.concatenate([zero, jnp.cumsum(n_e, dtype=jnp.int32)])
    total = start[E]
    ii = jnp.arange(G, dtype=jnp.int32)
    tile_e = jnp.sum((ii[:, None] >= start[None, 1:E + 1]).astype(jnp.int32),
                     axis=1)
    valid = ii < total
    tile_e = jnp.where(valid, jnp.minimum(tile_e, E - 1), E - 1)
    tile_m = jnp.where(valid, m_first[tile_e] + (ii - start[tile_e]), NB - 1)
    ts = jnp.where(valid, jnp.maximum(offsets[tile_e], tile_m * B), 0)
    tn = jnp.where(valid, jnp.minimum(offsets[tile_e + 1], (tile_m + 1) * B), 0)
    first = jnp.concatenate([jnp.ones((1,), jnp.int32),
                             (tile_m[1:] != tile_m[:-1]).astype(jnp.int32)])
    return tile_m, tile_e, ts, tn, first


def kernel(x, Wg, bg, W1, b1, W2, b2, W3, b3):
    eidx, rank, counts, offs = _gate(x, Wg, bg.reshape(1, E))
    e_flat = eidx.reshape(T)
    c_flat = counts.reshape(E)
    xs, p = _make_sc_dispatch()(x, e_flat, rank.reshape(T), offs.reshape(E))
    tile_m, tile_e, ts, tn, first = _tile_metadata(c_flat)
    o = _mlp(tile_m, tile_e, ts, tn, first,
             xs, W1, b1.reshape(E, 1, H), W2, b2.reshape(E, 1, H), W3)
    out = _make_sc_combine()(o.reshape(T), p, e_flat, b3.reshape(E))
    return out.reshape(T, 1)


# speedup vs baseline: 1.1172x; 1.0361x over previous
import functools

import jax
import jax.numpy as jnp
from jax import lax
from jax.experimental import pallas as pl
from jax.experimental.pallas import tpu as pltpu
from jax.experimental.pallas import tpu_sc as plsc

T, D, E, H = 8192, 768, 16, 128
TB = 1024
B = 512
NB = T // B
G = NB + E
NC, NS = 2, 16
NW = NC * NS
PW = T // NW
CH = 128


def _gate_body(x_ref, wg_ref, bg_ref, eidx_ref, rank_ref, cnt_ref, off_ref,
               acc_ref):
    i = pl.program_id(0)

    @pl.when(i == 0)
    def _():
        acc_ref[...] = jnp.zeros_like(acc_ref)

    logits = lax.dot_general(
        x_ref[...], wg_ref[...], (((1,), (1,)), ((), ())),
        preferred_element_type=jnp.float32,
    ) + bg_ref[...]
    m = jnp.max(logits, axis=1, keepdims=True)
    iota_e = lax.broadcasted_iota(jnp.int32, (TB, E), 1)
    eidx = jnp.min(jnp.where(logits == m, iota_e, E), axis=1, keepdims=True)
    onehot = (iota_e == eidx).astype(jnp.float32)

    prev = acc_ref[...]
    rank_prev = jnp.sum(onehot * prev, axis=1, keepdims=True)
    r_iota = lax.broadcasted_iota(jnp.int32, (TB, TB), 0)
    c_iota = lax.broadcasted_iota(jnp.int32, (TB, TB), 1)
    tri = (c_iota < r_iota).astype(jnp.float32)
    cnt_in = lax.dot_general(
        tri, onehot, (((1,), (0,)), ((), ())),
        preferred_element_type=jnp.float32,
    )
    rank_in = jnp.sum(cnt_in * onehot, axis=1, keepdims=True)

    eidx_ref[...] = eidx
    rank_ref[...] = (rank_prev + rank_in).astype(jnp.int32)
    new = prev + jnp.sum(onehot, axis=0, keepdims=True)
    acc_ref[...] = new
    cnt_ref[...] = new.astype(jnp.int32)
    r16 = lax.broadcasted_iota(jnp.int32, (E, E), 0)
    c16 = lax.broadcasted_iota(jnp.int32, (E, E), 1)
    upper = (r16 < c16).astype(jnp.float32)
    off_ref[...] = lax.dot_general(
        new, upper, (((1,), (0,)), ((), ())),
        preferred_element_type=jnp.float32,
        precision=lax.Precision.HIGHEST,
    ).astype(jnp.int32)


_gate = pl.pallas_call(
    _gate_body,
    grid=(T // TB,),
    in_specs=[
        pl.BlockSpec((TB, D), lambda i: (i, 0)),
        pl.BlockSpec((E, D), lambda i: (0, 0)),
        pl.BlockSpec((1, E), lambda i: (0, 0)),
    ],
    out_specs=[
        pl.BlockSpec((TB, 1), lambda i: (i, 0)),
        pl.BlockSpec((TB, 1), lambda i: (i, 0)),
        pl.BlockSpec((1, E), lambda i: (0, 0)),
        pl.BlockSpec((1, E), lambda i: (0, 0)),
    ],
    out_shape=[
        jax.ShapeDtypeStruct((T, 1), jnp.int32),
        jax.ShapeDtypeStruct((T, 1), jnp.int32),
        jax.ShapeDtypeStruct((1, E), jnp.int32),
        jax.ShapeDtypeStruct((1, E), jnp.int32),
    ],
    scratch_shapes=[pltpu.VMEM((1, E), jnp.float32)],
)


@functools.lru_cache(maxsize=None)
def _make_sc_dispatch():
    mesh = plsc.VectorSubcoreMesh(core_axis_name="c", subcore_axis_name="s",
                                  num_cores=NC, num_subcores=NS)

    @functools.partial(
        pl.kernel,
        out_type=[
            jax.ShapeDtypeStruct((T, D), jnp.float32),
            jax.ShapeDtypeStruct((T,), jnp.int32),
        ],
        mesh=mesh,
        scratch_types=[
            pltpu.VMEM((E,), jnp.int32),
            pltpu.VMEM((CH,), jnp.int32),
            pltpu.VMEM((CH,), jnp.int32),
            pltpu.VMEM((CH, D), jnp.float32),
            pltpu.SemaphoreType.DMA,
        ],
        compiler_params=pltpu.CompilerParams(needs_layout_passes=False),
    )
    def _sc_dispatch(x_hbm, e_hbm, r_hbm, off_hbm, xs_hbm, p_hbm,
                     off_v, e_v, p_v, rows_v, sem):
        wid = lax.axis_index("s") * NC + lax.axis_index("c")
        base = wid * PW
        pltpu.sync_copy(off_hbm, off_v)
        for c in range(PW // CH):
            cbase = base + c * CH
            pltpu.sync_copy(e_hbm.at[pl.ds(cbase, CH)], e_v)
            pltpu.sync_copy(r_hbm.at[pl.ds(cbase, CH)], p_v)
            for j in range(CH // 16):
                sl = pl.ds(j * 16, 16)
                p_v[sl] = p_v[sl] + plsc.load_gather(off_v, [e_v[sl]])
            pltpu.sync_copy(x_hbm.at[pl.ds(cbase, CH)], rows_v)
            pltpu.async_copy(rows_v, xs_hbm.at[p_v], sem).wait()
            pltpu.sync_copy(p_v, p_hbm.at[pl.ds(cbase, CH)])

    return _sc_dispatch


def _mlp_body(tm_ref, te_ref, ts_ref, tn_ref, first_ref,
              xs_ref, w1_ref, b1_ref, w2_ref, b2_ref, w3_ref, out_ref):
    i = pl.program_id(0)
    h1 = jnp.maximum(
        lax.dot_general(
            xs_ref[...], w1_ref[0], (((1,), (1,)), ((), ())),
            preferred_element_type=jnp.float32,
        ) + b1_ref[0], 0.0)
    h2 = jnp.maximum(
        lax.dot_general(
            h1, w2_ref[0], (((1,), (1,)), ((), ())),
            preferred_element_type=jnp.float32,
        ) + b2_ref[0], 0.0)
    o = jnp.sum(h2 * w3_ref[0], axis=1, keepdims=True)
    gidx = tm_ref[i] * B + lax.broadcasted_iota(jnp.int32, (B, 1), 0)
    val = jnp.where((gidx >= ts_ref[i]) & (gidx < tn_ref[i]), o, 0.0)

    @pl.when(first_ref[i] != 0)
    def _():
        out_ref[...] = val

    @pl.when(first_ref[i] == 0)
    def _():
        out_ref[...] = out_ref[...] + val


_mlp = pl.pallas_call(
    _mlp_body,
    grid_spec=pltpu.PrefetchScalarGridSpec(
        num_scalar_prefetch=5,
        grid=(G,),
        in_specs=[
            pl.BlockSpec((B, D), lambda i, *r: (r[0][i], 0)),
            pl.BlockSpec((1, H, D), lambda i, *r: (r[1][i], 0, 0)),
            pl.BlockSpec((1, 1, H), lambda i, *r: (r[1][i], 0, 0)),
            pl.BlockSpec((1, H, H), lambda i, *r: (r[1][i], 0, 0)),
            pl.BlockSpec((1, 1, H), lambda i, *r: (r[1][i], 0, 0)),
            pl.BlockSpec((1, 1, H), lambda i, *r: (r[1][i], 0, 0)),
        ],
        out_specs=pl.BlockSpec((B, 1), lambda i, *r: (r[0][i], 0)),
    ),
    out_shape=jax.ShapeDtypeStruct((T, 1), jnp.float32),
)


@functools.lru_cache(maxsize=None)
def _make_sc_combine():
    mesh = plsc.VectorSubcoreMesh(core_axis_name="c", subcore_axis_name="s",
                                  num_cores=NC, num_subcores=NS)

    @functools.partial(
        pl.kernel,
        out_type=jax.ShapeDtypeStruct((T,), jnp.float32),
        mesh=mesh,
        scratch_types=[
            pltpu.VMEM((T,), jnp.float32),
            pltpu.VMEM((E,), jnp.float32),
            pltpu.VMEM((PW,), jnp.int32),
            pltpu.VMEM((PW,), jnp.int32),
            pltpu.VMEM((PW,), jnp.float32),
        ],
        compiler_params=pltpu.CompilerParams(needs_layout_passes=False),
    )
    def _sc_combine(o_hbm, p_hbm, e_hbm, b3_hbm, out_hbm,
                    o_v, b3_v, p_v, e_v, res_v):
        wid = lax.axis_index("s") * NC + lax.axis_index("c")
        base = wid * PW
        pltpu.sync_copy(o_hbm, o_v)
        pltpu.sync_copy(b3_hbm, b3_v)
        pltpu.sync_copy(p_hbm.at[pl.ds(base, PW)], p_v)
        pltpu.sync_copy(e_hbm.at[pl.ds(base, PW)], e_v)
        for j in range(PW // 16):
            sl = pl.ds(j * 16, 16)
            res_v[sl] = (plsc.load_gather(o_v, [p_v[sl]])
                         + plsc.load_gather(b3_v, [e_v[sl]]))
        pltpu.sync_copy(res_v, out_hbm.at[pl.ds(base, PW)])

    return _sc_combine


def _tile_metadata(counts):
    zero = jnp.zeros((1,), jnp.int32)
    offsets = jnp.concatenate([zero, jnp.cumsum(counts, dtype=jnp.int32)])
    m_first = offsets[:E] // B
    m_last = jnp.maximum(offsets[1:] - 1, 0) // B
    n_e = jnp.where(counts > 0, m_last - m_first + 1, 0).astype(jnp.int32)
    start = jnp.concatenate([zero, jnp.cumsum(n_e, dtype=jnp.int32)])
    total = start[E]
    ii = jnp.arange(G, dtype=jnp.int32)
    tile_e = jnp.sum((ii[:, None] >= start[None, 1:E + 1]).astype(jnp.int32),
                     axis=1)
    valid = ii < total
    tile_e = jnp.where(valid, jnp.minimum(tile_e, E - 1), E - 1)
    tile_m = jnp.where(valid, m_first[tile_e] + (ii - start[tile_e]), NB - 1)
    ts = jnp.where(valid, jnp.maximum(offsets[tile_e], tile_m * B), 0)
    tn = jnp.where(valid, jnp.minimum(offsets[tile_e + 1], (tile_m + 1) * B), 0)
    first = jnp.concatenate([jnp.ones((1,), jnp.int32),
                             (tile_m[1:] != tile_m[:-1]).astype(jnp.int32)])
    return tile_m, tile_e, ts, tn, first


def kernel(x, Wg, bg, W1, b1, W2, b2, W3, b3):
    eidx, rank, counts, offs = _gate(x, Wg, bg.reshape(1, E))
    e_flat = eidx.reshape(T)
    c_flat = counts.reshape(E)
    xs, p = _make_sc_dispatch()(x, e_flat, rank.reshape(T), offs.reshape(E))
    tile_m, tile_e, ts, tn, first = _tile_metadata(c_flat)
    o = _mlp(tile_m, tile_e, ts, tn, first,
             xs, W1, b1.reshape(E, 1, H), W2, b2.reshape(E, 1, H), W3)
    out = _make_sc_combine()(o.reshape(T), p, e_flat, b3.reshape(E))
    return out.reshape(T, 1)
